# Initial kernel scaffold; baseline (speedup 1.0000x reference)
import jax
import jax.numpy as jnp
from jax.experimental import pallas as pl


def kernel(x, edge_index, W1l, b1, W1r, W2l, b2, W2r):
    raise NotImplementedError("write your pallas kernel here")


# speedup vs baseline: 4.1111x; 4.1111x over previous
import functools

import jax
import jax.numpy as jnp
from jax import lax
from jax.experimental import pallas as pl
from jax.experimental.pallas import tpu as pltpu
from jax.experimental.pallas import tpu_sc as plsc

N = 10000
D = 128
NP = 10240
NC = 2
NS = 16
NW = NC * NS
C = 64
ROWS_PER_TILE = NP // NS


def _dual_matmul_body(x_ref, wa_ref, wb_ref, a_ref, b_ref):
    xv = x_ref[...]
    dn = (((1,), (1,)), ((), ()))
    a_ref[...] = lax.dot_general(xv, wa_ref[...], dn, preferred_element_type=jnp.float32)
    b_ref[...] = lax.dot_general(xv, wb_ref[...], dn, preferred_element_type=jnp.float32)


def _tc_dual_matmul(x, wa, wb, br=2048):
    n = x.shape[0]
    return pl.pallas_call(
        _dual_matmul_body,
        grid=(n // br,),
        in_specs=[
            pl.BlockSpec((br, D), lambda i: (i, 0)),
            pl.BlockSpec((D, D), lambda i: (0, 0)),
            pl.BlockSpec((D, D), lambda i: (0, 0)),
        ],
        out_specs=[
            pl.BlockSpec((br, D), lambda i: (i, 0)),
            pl.BlockSpec((br, D), lambda i: (i, 0)),
        ],
        out_shape=[
            jax.ShapeDtypeStruct((n, D), jnp.float32),
            jax.ShapeDtypeStruct((n, D), jnp.float32),
        ],
    )(x, wa, wb)


def _mid_body(a0_ref, a1_ref, cnt_ref, r_ref, b_ref, wa_ref, wb_ref, g_ref, rr_ref):
    inv = 1.0 / jnp.maximum(cnt_ref[...], 1.0)
    mean = (a0_ref[...] + a1_ref[...]) * inv[:, None]
    h = jnp.maximum(mean + r_ref[...] + b_ref[...][None, :], 0.0)
    dn = (((1,), (1,)), ((), ()))
    g_ref[...] = lax.dot_general(h, wa_ref[...], dn, preferred_element_type=jnp.float32)
    rr_ref[...] = lax.dot_general(h, wb_ref[...], dn, preferred_element_type=jnp.float32)


def _tc_mid(agg0, agg1, cnt, r, b, wa, wb, br=2048):
    n = r.shape[0]
    return pl.pallas_call(
        _mid_body,
        grid=(n // br,),
        in_specs=[
            pl.BlockSpec((br, D), lambda i: (i, 0)),
            pl.BlockSpec((br, D), lambda i: (i, 0)),
            pl.BlockSpec((br,), lambda i: (i,)),
            pl.BlockSpec((br, D), lambda i: (i, 0)),
            pl.BlockSpec((D,), lambda i: (0,)),
            pl.BlockSpec((D, D), lambda i: (0, 0)),
            pl.BlockSpec((D, D), lambda i: (0, 0)),
        ],
        out_specs=[
            pl.BlockSpec((br, D), lambda i: (i, 0)),
            pl.BlockSpec((br, D), lambda i: (i, 0)),
        ],
        out_shape=[
            jax.ShapeDtypeStruct((n, D), jnp.float32),
            jax.ShapeDtypeStruct((n, D), jnp.float32),
        ],
    )(agg0, agg1, cnt, r, b, wa, wb)


def _final_body(a0_ref, a1_ref, cnt_ref, r_ref, b_ref, o_ref):
    inv = 1.0 / jnp.maximum(cnt_ref[...], 1.0)
    mean = (a0_ref[...] + a1_ref[...]) * inv[:, None]
    o_ref[...] = mean + r_ref[...] + b_ref[...][None, :]


def _tc_final(agg0, agg1, cnt, r, b, br=2048):
    n = r.shape[0]
    return pl.pallas_call(
        _final_body,
        grid=(n // br,),
        in_specs=[
            pl.BlockSpec((br, D), lambda i: (i, 0)),
            pl.BlockSpec((br, D), lambda i: (i, 0)),
            pl.BlockSpec((br,), lambda i: (i,)),
            pl.BlockSpec((br, D), lambda i: (i, 0)),
            pl.BlockSpec((D,), lambda i: (0,)),
        ],
        out_specs=pl.BlockSpec((br, D), lambda i: (i, 0)),
        out_shape=jax.ShapeDtypeStruct((n, D), jnp.float32),
    )(agg0, agg1, cnt, r, b)


def _sc_body(g_hbm, src_hbm, dst_hbm, agg_out,
             src_v, dst_v, rows_v, z_v, acc_sh, sem):
    cid = lax.axis_index("c")
    sid = lax.axis_index("s")
    wid = cid * NS + sid
    per_w = src_hbm.shape[0] // NW
    n_chunks = per_w // C
    r0 = sid * ROWS_PER_TILE

    def zfill(k, _):
        i = k // (D // 16)
        j = k % (D // 16)
        z_v[i, pl.ds(j * 16, 16)] = jnp.zeros((16,), jnp.float32)
        return 0
    lax.fori_loop(0, 16 * (D // 16), zfill, 0)

    def zinit(i, _):
        pltpu.sync_copy(z_v, acc_sh.at[pl.ds(r0 + i * 16, 16)])
        return 0
    lax.fori_loop(0, ROWS_PER_TILE // 16, zinit, 0)

    plsc.subcore_barrier()

    def step(i, _):
        base = wid * per_w + i * C
        pltpu.sync_copy(src_hbm.at[pl.ds(base, C)], src_v)
        pltpu.sync_copy(dst_hbm.at[pl.ds(base, C)], dst_v)
        pltpu.async_copy(g_hbm.at[src_v], rows_v, sem).wait()
        pltpu.sync_copy(rows_v, acc_sh.at[dst_v], add=True)
        return 0
    lax.fori_loop(0, n_chunks, step, 0)

    plsc.subcore_barrier()

    pltpu.sync_copy(acc_sh.at[pl.ds(r0, ROWS_PER_TILE)],
                    agg_out.at[pl.ds(cid * NP + r0, ROWS_PER_TILE)])


def _make_sc_segsum():
    mesh = plsc.VectorSubcoreMesh(core_axis_name="c", subcore_axis_name="s")
    out_type = jax.ShapeDtypeStruct((NC * NP, D), jnp.float32)
    scratch = [
        pltpu.VMEM((C,), jnp.int32),
        pltpu.VMEM((C,), jnp.int32),
        pltpu.VMEM((C, D), jnp.float32),
        pltpu.VMEM((16, D), jnp.float32),
        pltpu.VMEM_SHARED((NP, D), jnp.float32),
        pltpu.SemaphoreType.DMA,
    ]
    return pl.kernel(_sc_body, out_type=out_type, mesh=mesh, scratch_types=scratch)


def kernel(x, edge_index, W1l, b1, W1r, W2l, b2, W2r):
    E = edge_index.shape[1]
    per_w = -(-E // (NW * C)) * C
    e_pad = per_w * NW
    src = jnp.concatenate([edge_index[0], jnp.zeros((e_pad - E,), jnp.int32)])
    dst = jnp.concatenate([edge_index[1], jnp.full((e_pad - E,), N, jnp.int32)])
    x_p = jnp.pad(x, ((0, NP - N), (0, 0)))

    cnt = jax.ops.segment_sum(jnp.ones((E,), jnp.float32), edge_index[1],
                              num_segments=NP)

    sc_segsum = _make_sc_segsum()

    g1, r1 = _tc_dual_matmul(x_p, W1l, W1r)
    agg1p = sc_segsum(g1, src, dst)
    g2, r2 = _tc_mid(agg1p[:NP], agg1p[NP:], cnt, r1, b1, W2l, W2r)
    agg2p = sc_segsum(g2, src, dst)
    out = _tc_final(agg2p[:NP], agg2p[NP:], cnt, r2, b2)
    return out[:N]
